# baseline (speedup 1.0000x reference)
import functools

import jax
import jax.numpy as jnp
from jax import lax
from jax.experimental import pallas as pl
from jax.experimental.pallas import tpu as pltpu
from jax.experimental.pallas import tpu_sc as plsc

ROW = 7
L = 16
NW = 32
NC = 2


def _sc_pose_correction(n_rows, batch):
    b_per_w = batch // NW
    groups = b_per_w // L
    mesh = plsc.VectorSubcoreMesh(core_axis_name="c", subcore_axis_name="s")

    @functools.partial(
        pl.kernel,
        mesh=mesh,
        compiler_params=pltpu.CompilerParams(needs_layout_passes=False),
        out_type=jax.ShapeDtypeStruct((batch * 6,), jnp.float32),
        scratch_types=[
            pltpu.VMEM((n_rows * ROW,), jnp.float32),
            pltpu.VMEM((b_per_w,), jnp.int32),
            pltpu.VMEM((b_per_w,), jnp.int32),
            pltpu.VMEM((b_per_w * 6,), jnp.float32),
            pltpu.VMEM((b_per_w * 6,), jnp.float32),
            pltpu.SemaphoreType.DMA,
            pltpu.SemaphoreType.DMA,
            pltpu.SemaphoreType.DMA,
            pltpu.SemaphoreType.DMA,
        ],
    )
    def k(table_hbm, idx_hbm, mask_hbm, rays_hbm, out_hbm,
          table_v, idx_v, mask_v, rays_v, out_v,
          sem_t, sem_i, sem_m, sem_r):
        wid = lax.axis_index("s") * NC + lax.axis_index("c")
        base = wid * b_per_w
        cp_t = pltpu.async_copy(table_hbm, table_v, sem_t)
        cp_i = pltpu.async_copy(idx_hbm.at[pl.ds(base, b_per_w)], idx_v, sem_i)
        cp_m = pltpu.async_copy(mask_hbm.at[pl.ds(base, b_per_w)], mask_v, sem_m)
        cp_r = pltpu.async_copy(
            rays_hbm.at[pl.ds(base * 6, b_per_w * 6)], rays_v, sem_r)
        cp_i.wait()
        cp_m.wait()
        cp_r.wait()
        cp_t.wait()

        zeros = jnp.zeros((L,), jnp.float32)
        ones = jnp.ones((L,), jnp.float32)
        lane = lax.iota(jnp.int32, L)

        def body(g, carry):
            sl = pl.ds(g * L, L)
            idx = idx_v[sl] * ROW
            m = mask_v[sl] == 1
            ray6 = (g * L) * 6 + lane * 6

            def gat(c, ident):
                return jnp.where(m, plsc.load_gather(table_v, [idx + c]), ident)

            def ray(c):
                return plsc.load_gather(rays_v, [ray6 + c])

            tx = gat(0, zeros)
            ty = gat(1, zeros)
            tz = gat(2, zeros)
            qx = gat(3, zeros)
            qy = gat(4, zeros)
            qz = gat(5, zeros)
            qw = gat(6, ones)

            dx = ray(3)
            dy = ray(4)
            dz = ray(5)

            r00 = 1.0 - 2.0 * (qy * qy + qz * qz)
            r01 = 2.0 * (qx * qy - qz * qw)
            r02 = 2.0 * (qx * qz + qy * qw)
            r10 = 2.0 * (qx * qy + qz * qw)
            r11 = 1.0 - 2.0 * (qx * qx + qz * qz)
            r12 = 2.0 * (qy * qz - qx * qw)
            r20 = 2.0 * (qx * qz - qy * qw)
            r21 = 2.0 * (qy * qz + qx * qw)
            r22 = 1.0 - 2.0 * (qx * qx + qy * qy)

            plsc.store_scatter(out_v, [ray6 + 0], ray(0) + tx)
            plsc.store_scatter(out_v, [ray6 + 1], ray(1) + ty)
            plsc.store_scatter(out_v, [ray6 + 2], ray(2) + tz)
            plsc.store_scatter(out_v, [ray6 + 3], r00 * dx + r01 * dy + r02 * dz)
            plsc.store_scatter(out_v, [ray6 + 4], r10 * dx + r11 * dy + r12 * dz)
            plsc.store_scatter(out_v, [ray6 + 5], r20 * dx + r21 * dy + r22 * dz)
            return carry

        lax.fori_loop(0, groups, body, 0)
        pltpu.sync_copy(out_v, out_hbm.at[pl.ds(base * 6, b_per_w * 6)])

    return k


def kernel(image_indices, rays, depth_mask, correction_dict):
    batch = rays.shape[0]
    n_rows = correction_dict.shape[0]

    table = correction_dict.reshape(-1)
    idx = image_indices.astype(jnp.int32)
    mask = depth_mask.astype(jnp.int32).reshape(-1)
    rays_flat = rays.reshape(-1)

    out = _sc_pose_correction(n_rows, batch)(table, idx, mask, rays_flat)
    return out.reshape(batch, 6)


# speedup vs baseline: 2.2230x; 2.2230x over previous
import functools

import jax
import jax.numpy as jnp
from jax import lax
from jax.experimental import pallas as pl
from jax.experimental.pallas import tpu as pltpu
from jax.experimental.pallas import tpu_sc as plsc

ROW = 8
L = 16
NW = 32
NC = 2


def _sc_pose_correction(n_rows, batch):
    b_per_w = batch // NW
    groups = b_per_w // L
    mesh = plsc.VectorSubcoreMesh(core_axis_name="c", subcore_axis_name="s")

    @functools.partial(
        pl.kernel,
        mesh=mesh,
        compiler_params=pltpu.CompilerParams(needs_layout_passes=False),
        out_type=jax.ShapeDtypeStruct((6, batch), jnp.float32),
        scratch_types=[
            pltpu.VMEM((n_rows * ROW,), jnp.float32),
            pltpu.VMEM((b_per_w,), jnp.int32),
            pltpu.VMEM((b_per_w,), jnp.int32),
            pltpu.VMEM((6, b_per_w), jnp.float32),
            pltpu.VMEM((6, b_per_w), jnp.float32),
        ],
    )
    def k(table_hbm, idx_hbm, mask_hbm, rays_hbm, out_hbm,
          table_v, idx_v, mask_v, rays_v, out_v):
        wid = lax.axis_index("s") * NC + lax.axis_index("c")
        base = wid * b_per_w
        pltpu.sync_copy(table_hbm, table_v)
        pltpu.sync_copy(idx_hbm.at[pl.ds(base, b_per_w)], idx_v)
        pltpu.sync_copy(mask_hbm.at[pl.ds(base, b_per_w)], mask_v)
        pltpu.sync_copy(rays_hbm.at[:, pl.ds(base, b_per_w)], rays_v)

        zeros = jnp.zeros((L,), jnp.float32)
        ones = jnp.ones((L,), jnp.float32)

        def body(g, carry):
            sl = pl.ds(g * L, L)
            idx = idx_v[sl] * ROW
            m = mask_v[sl] == 1

            def gat(c, ident):
                return jnp.where(m, plsc.load_gather(table_v, [idx + c]), ident)

            tx = gat(0, zeros)
            ty = gat(1, zeros)
            tz = gat(2, zeros)
            qx = gat(3, zeros)
            qy = gat(4, zeros)
            qz = gat(5, zeros)
            qw = gat(6, ones)

            dx = rays_v[3, sl]
            dy = rays_v[4, sl]
            dz = rays_v[5, sl]

            r00 = 1.0 - 2.0 * (qy * qy + qz * qz)
            r01 = 2.0 * (qx * qy - qz * qw)
            r02 = 2.0 * (qx * qz + qy * qw)
            r10 = 2.0 * (qx * qy + qz * qw)
            r11 = 1.0 - 2.0 * (qx * qx + qz * qz)
            r12 = 2.0 * (qy * qz - qx * qw)
            r20 = 2.0 * (qx * qz - qy * qw)
            r21 = 2.0 * (qy * qz + qx * qw)
            r22 = 1.0 - 2.0 * (qx * qx + qy * qy)

            out_v[0, sl] = rays_v[0, sl] + tx
            out_v[1, sl] = rays_v[1, sl] + ty
            out_v[2, sl] = rays_v[2, sl] + tz
            out_v[3, sl] = r00 * dx + r01 * dy + r02 * dz
            out_v[4, sl] = r10 * dx + r11 * dy + r12 * dz
            out_v[5, sl] = r20 * dx + r21 * dy + r22 * dz
            return carry

        lax.fori_loop(0, groups, body, 0)
        pltpu.sync_copy(out_v, out_hbm.at[:, pl.ds(base, b_per_w)])

    return k


def kernel(image_indices, rays, depth_mask, correction_dict):
    batch = rays.shape[0]
    n_rows = correction_dict.shape[0]

    table = jnp.concatenate(
        [correction_dict,
         jnp.zeros((n_rows, ROW - correction_dict.shape[1]), correction_dict.dtype)],
        axis=1,
    ).reshape(-1)
    idx = image_indices.astype(jnp.int32)
    mask = depth_mask.astype(jnp.int32).reshape(-1)
    rays_t = rays.T

    out = _sc_pose_correction(n_rows, batch)(table, idx, mask, rays_t)
    return out.T


# speedup vs baseline: 2.2947x; 1.0322x over previous
import functools

import jax
import jax.numpy as jnp
from jax import lax
from jax.experimental import pallas as pl
from jax.experimental.pallas import tpu as pltpu
from jax.experimental.pallas import tpu_sc as plsc

L = 16
NW = 32
NC = 2


def _sc_pose_correction(n_rows, batch):
    b_per_w = batch // NW
    groups = b_per_w // L
    mesh = plsc.VectorSubcoreMesh(core_axis_name="c", subcore_axis_name="s")

    @functools.partial(
        pl.kernel,
        mesh=mesh,
        compiler_params=pltpu.CompilerParams(needs_layout_passes=False),
        out_type=jax.ShapeDtypeStruct((6, batch), jnp.float32),
        scratch_types=[
            pltpu.VMEM((7, n_rows), jnp.float32),
            pltpu.VMEM((b_per_w,), jnp.int32),
            pltpu.VMEM((b_per_w,), jnp.int32),
            pltpu.VMEM((6, b_per_w), jnp.float32),
            pltpu.VMEM((6, b_per_w), jnp.float32),
            pltpu.SemaphoreType.DMA,
            pltpu.SemaphoreType.DMA,
            pltpu.SemaphoreType.DMA,
            pltpu.SemaphoreType.DMA,
        ],
    )
    def k(table_hbm, idx_hbm, mask_hbm, rays_hbm, out_hbm,
          table_v, idx_v, mask_v, rays_v, out_v,
          sem_t, sem_i, sem_m, sem_r):
        wid = lax.axis_index("s") * NC + lax.axis_index("c")
        base = wid * b_per_w
        cps = [pltpu.async_copy(table_hbm, table_v, sem_t)]
        cps.append(pltpu.async_copy(
            idx_hbm.at[pl.ds(base, b_per_w)], idx_v, sem_i))
        cps.append(pltpu.async_copy(
            mask_hbm.at[pl.ds(base, b_per_w)], mask_v, sem_m))
        cps.append(pltpu.async_copy(
            rays_hbm.at[:, pl.ds(base, b_per_w)], rays_v, sem_r))
        for cp in cps:
            cp.wait()

        zeros = jnp.zeros((L,), jnp.float32)
        ones = jnp.ones((L,), jnp.float32)

        def body(g, carry):
            sl = pl.ds(g * L, L)
            idx = idx_v[sl]
            m = mask_v[sl] == 1

            def gat(c, ident):
                col = jnp.full((L,), c, jnp.int32)
                return jnp.where(m, plsc.load_gather(table_v, [col, idx]), ident)

            tx = gat(0, zeros)
            ty = gat(1, zeros)
            tz = gat(2, zeros)
            qx = gat(3, zeros)
            qy = gat(4, zeros)
            qz = gat(5, zeros)
            qw = gat(6, ones)

            dx = rays_v[3, sl]
            dy = rays_v[4, sl]
            dz = rays_v[5, sl]

            r00 = 1.0 - 2.0 * (qy * qy + qz * qz)
            r01 = 2.0 * (qx * qy - qz * qw)
            r02 = 2.0 * (qx * qz + qy * qw)
            r10 = 2.0 * (qx * qy + qz * qw)
            r11 = 1.0 - 2.0 * (qx * qx + qz * qz)
            r12 = 2.0 * (qy * qz - qx * qw)
            r20 = 2.0 * (qx * qz - qy * qw)
            r21 = 2.0 * (qy * qz + qx * qw)
            r22 = 1.0 - 2.0 * (qx * qx + qy * qy)

            out_v[0, sl] = rays_v[0, sl] + tx
            out_v[1, sl] = rays_v[1, sl] + ty
            out_v[2, sl] = rays_v[2, sl] + tz
            out_v[3, sl] = r00 * dx + r01 * dy + r02 * dz
            out_v[4, sl] = r10 * dx + r11 * dy + r12 * dz
            out_v[5, sl] = r20 * dx + r21 * dy + r22 * dz
            return carry

        lax.fori_loop(0, groups, body, 0)
        pltpu.sync_copy(out_v, out_hbm.at[:, pl.ds(base, b_per_w)])

    return k


def kernel(image_indices, rays, depth_mask, correction_dict):
    batch = rays.shape[0]
    n_rows = correction_dict.shape[0]

    table_t = correction_dict.T
    idx = image_indices.astype(jnp.int32)
    mask = depth_mask.astype(jnp.int32).reshape(-1)
    rays_t = rays.T

    out = _sc_pose_correction(n_rows, batch)(table_t, idx, mask, rays_t)
    return out.T
